# Initial kernel scaffold; baseline (speedup 1.0000x reference)
import jax
import jax.numpy as jnp
from jax.experimental import pallas as pl


def kernel(x, edge_index, Wl1_artist, Wr1_artist, b1_artist, Wl2_artist, Wr2_artist, b2_artist, Wl1_style, Wr1_style, b1_style, Wl2_style, Wr2_style, b2_style, Wl1_genre, Wr1_genre, b1_genre, Wl2_genre, Wr2_genre, b2_genre):
    raise NotImplementedError("write your pallas kernel here")


# speedup vs baseline: 7.8285x; 7.8285x over previous
import functools

import jax
import jax.numpy as jnp
from jax import lax
from jax.experimental import pallas as pl
from jax.experimental.pallas import tpu as pltpu
from jax.experimental.pallas import tpu_sc as plsc

_N = 10000
_E = 320000
_D = 128
_W1 = 144
_W2 = 152
_NC = 2
_NS = 16
_NW = _NC * _NS
_EPW = _E // _NW
_CH = 80
_NCH = _EPW // _CH
_NP = 10000
_RPS = _NP // _NS

_OUTS = (100, 30, 15)
_OFFS = (0, 100, 130, 145)


def _make_segsum(width):
    mesh = plsc.VectorSubcoreMesh(core_axis_name="c", subcore_axis_name="s",
                                  num_cores=_NC, num_subcores=_NS)

    @functools.partial(
        pl.kernel,
        out_type=jax.ShapeDtypeStruct((_NC, _NP, width), jnp.float32),
        mesh=mesh,
        scratch_types=[
            pltpu.VMEM((_NCH, _CH), jnp.int32),
            pltpu.VMEM((_NCH, _CH), jnp.int32),
            pltpu.VMEM((_CH, width), jnp.float32),
            pltpu.VMEM_SHARED((_NP, width), jnp.float32),
            pltpu.SemaphoreType.DMA,
        ],
        compiler_params=pltpu.CompilerParams(use_tc_tiling_on_sc=False),
    )
    def segsum(table_hbm, src_hbm, dst_hbm, zeros_hbm, out_hbm,
               src_v, dst_v, rows_v, accum, sem):
        c = lax.axis_index("c")
        s = lax.axis_index("s")
        wid = c * _NS + s
        pltpu.sync_copy(zeros_hbm.at[pl.ds(s * _RPS, _RPS)],
                        accum.at[pl.ds(s * _RPS, _RPS)])
        pltpu.sync_copy(src_hbm.at[wid], src_v)
        pltpu.sync_copy(dst_hbm.at[wid], dst_v)
        plsc.subcore_barrier()

        def body(j, carry):
            pltpu.async_copy(table_hbm.at[src_v.at[j]], rows_v, sem).wait()
            pltpu.sync_copy(rows_v, accum.at[dst_v.at[j]], add=True)
            return carry

        lax.fori_loop(0, _NCH, body, 0)
        plsc.subcore_barrier()
        pltpu.sync_copy(accum.at[pl.ds(s * _RPS, _RPS)],
                        out_hbm.at[c, pl.ds(s * _RPS, _RPS)])

    return segsum


_segsum_x = _make_segsum(_W1)
_segsum_g = _make_segsum(_W2)

_BLK = 1000
_GRID = _N // _BLK


def _dense1_body(agg_ref, x_ref, w1_ref, wr1_ref, b1_ref, w2_ref, wr2_ref,
                 g_ref, r_ref, inv_ref):
    agg = agg_ref[0] + agg_ref[1]
    cnt = agg[:, _D:_D + 1]
    inv = 1.0 / jnp.maximum(cnt, 1.0)
    mean = agg[:, :_D] * inv
    pre = (jnp.dot(mean, w1_ref[...], preferred_element_type=jnp.float32)
           + jnp.dot(x_ref[...], wr1_ref[...], preferred_element_type=jnp.float32)
           + b1_ref[...])
    h = jnp.maximum(pre, 0.0)
    g_ref[...] = jnp.dot(h, w2_ref[...], preferred_element_type=jnp.float32)
    r_ref[...] = jnp.dot(h, wr2_ref[...], preferred_element_type=jnp.float32)
    inv_ref[...] = inv


def _dense2_body(agg_ref, r_ref, inv_ref, b2_ref,
                 oa_ref, os_ref, og_ref):
    ag = agg_ref[0] + agg_ref[1]
    inv = inv_ref[...]
    o = ag * inv + r_ref[...] + b2_ref[...]
    lane = lax.broadcasted_iota(jnp.int32, o.shape, 1)
    neg = jnp.float32(-1e30)
    logden = jnp.zeros_like(o)
    mx = jnp.zeros_like(o)
    for i in range(3):
        m = (lane >= _OFFS[i]) & (lane < _OFFS[i + 1])
        mh = jnp.max(jnp.where(m, o, neg), axis=1, keepdims=True)
        mx = jnp.where(m, mh, mx)
        sh = jnp.sum(jnp.where(m, jnp.exp(o - mh), 0.0), axis=1, keepdims=True)
        logden = jnp.where(m, jnp.log(sh), logden)
    out = o - mx - logden
    oa_ref[...] = out[:, 0:100]
    os_ref[...] = out[:, 100:130]
    og_ref[...] = out[:, 130:145]


def _bcast_spec(shape):
    nd = len(shape)
    return pl.BlockSpec(shape, lambda i, _nd=nd: (0,) * _nd)


def _dense1(agg1, x, w1, wr1, b1, w2, wr2):
    return pl.pallas_call(
        _dense1_body,
        grid=(_GRID,),
        in_specs=[
            pl.BlockSpec((_NC, _BLK, _W1), lambda i: (0, i, 0)),
            pl.BlockSpec((_BLK, _D), lambda i: (i, 0)),
            _bcast_spec((_D, 384)),
            _bcast_spec((_D, 384)),
            _bcast_spec((1, 384)),
            _bcast_spec((384, _W2)),
            _bcast_spec((384, _W2)),
        ],
        out_specs=[
            pl.BlockSpec((_BLK, _W2), lambda i: (i, 0)),
            pl.BlockSpec((_BLK, _W2), lambda i: (i, 0)),
            pl.BlockSpec((_BLK, 1), lambda i: (i, 0)),
        ],
        out_shape=[
            jax.ShapeDtypeStruct((_N, _W2), jnp.float32),
            jax.ShapeDtypeStruct((_N, _W2), jnp.float32),
            jax.ShapeDtypeStruct((_N, 1), jnp.float32),
        ],
    )(agg1, x, w1, wr1, b1, w2, wr2)


def _dense2(agg2, r, inv, b2):
    return pl.pallas_call(
        _dense2_body,
        grid=(_GRID,),
        in_specs=[
            pl.BlockSpec((_NC, _BLK, _W2), lambda i: (0, i, 0)),
            pl.BlockSpec((_BLK, _W2), lambda i: (i, 0)),
            pl.BlockSpec((_BLK, 1), lambda i: (i, 0)),
            _bcast_spec((1, _W2)),
        ],
        out_specs=[
            pl.BlockSpec((_BLK, 100), lambda i: (i, 0)),
            pl.BlockSpec((_BLK, 30), lambda i: (i, 0)),
            pl.BlockSpec((_BLK, 15), lambda i: (i, 0)),
        ],
        out_shape=[
            jax.ShapeDtypeStruct((_N, 100), jnp.float32),
            jax.ShapeDtypeStruct((_N, 30), jnp.float32),
            jax.ShapeDtypeStruct((_N, 15), jnp.float32),
        ],
    )(agg2, r, inv, b2)


def kernel(x, edge_index,
           Wl1_artist, Wr1_artist, b1_artist, Wl2_artist, Wr2_artist, b2_artist,
           Wl1_style, Wr1_style, b1_style, Wl2_style, Wr2_style, b2_style,
           Wl1_genre, Wr1_genre, b1_genre, Wl2_genre, Wr2_genre, b2_genre):
    src = edge_index[0].reshape(_NW, _NCH, _CH)
    dst = edge_index[1].reshape(_NW, _NCH, _CH)
    x_pad = jnp.concatenate(
        [x, jnp.ones((_N, 1), jnp.float32), jnp.zeros((_N, _W1 - _D - 1), jnp.float32)],
        axis=1)

    w1 = jnp.concatenate([Wl1_artist.T, Wl1_style.T, Wl1_genre.T], axis=1)
    wr1 = jnp.concatenate([Wr1_artist.T, Wr1_style.T, Wr1_genre.T], axis=1)
    b1 = jnp.concatenate([b1_artist, b1_style, b1_genre]).reshape(1, 384)

    def blockdiag(wa, ws, wg):
        z = jnp.zeros((384, _W2), jnp.float32)
        z = z.at[0:_D, 0:100].set(wa.T)
        z = z.at[_D:2 * _D, 100:130].set(ws.T)
        z = z.at[2 * _D:3 * _D, 130:145].set(wg.T)
        return z

    w2 = blockdiag(Wl2_artist, Wl2_style, Wl2_genre)
    wr2 = blockdiag(Wr2_artist, Wr2_style, Wr2_genre)
    b2 = jnp.zeros((1, _W2), jnp.float32)
    b2 = b2.at[0, 0:100].set(b2_artist)
    b2 = b2.at[0, 100:130].set(b2_style)
    b2 = b2.at[0, 130:145].set(b2_genre)

    agg1 = _segsum_x(x_pad, src, dst, jnp.zeros((_NP, _W1), jnp.float32))
    g, r, inv = _dense1(agg1, x, w1, wr1, b1, w2, wr2)
    agg2 = _segsum_g(g, src, dst, jnp.zeros((_NP, _W2), jnp.float32))
    out_a, out_s, out_g = _dense2(agg2, r, inv, b2)
    return (out_a, out_s, out_g)
